# baseline (speedup 1.0000x reference)
import functools

import jax
import jax.numpy as jnp
from jax import lax
from jax.experimental import pallas as pl
from jax.experimental.pallas import tpu as pltpu
from jax.experimental.pallas import tpu_sc as plsc

N = 10000
E = 320000
D = 128

NC = 2
NS = 16
NW = NC * NS
CH = 128
NCHUNK = 2 * (-(-E // (NW * CH * 2)))
EPW = NCHUNK * CH
EP = EPW * NW
NCH0 = 46
NCH1 = 2 * NCHUNK - NCH0
NP = -(-(N + 1) // (NS * 8)) * NS * 8
RPT = NP // NS
_CHUNKS = [(o, min(CH, RPT - o)) for o in range(0, RPT, CH)]

_MESH = plsc.VectorSubcoreMesh(core_axis_name="c", subcore_axis_name="s")


@functools.partial(
    pl.kernel, mesh=_MESH,
    out_type=jax.ShapeDtypeStruct((NC, NP, D), jnp.float32),
    scratch_types=[
        pltpu.VMEM_SHARED((NP, D), jnp.float32),
        pltpu.VMEM((CH, D), jnp.float32),
        pltpu.VMEM((CH, D), jnp.float32),
        pltpu.VMEM((CH,), jnp.int32),
        pltpu.VMEM((CH,), jnp.int32),
        pltpu.VMEM((CH,), jnp.int32),
        pltpu.VMEM((CH,), jnp.int32),
        pltpu.SemaphoreType.DMA,
        pltpu.SemaphoreType.DMA,
    ])
def _sc_agg(table_hbm, src_hbm, dst_hbm, zeros_hbm, agg_out,
            acc_sh, rows0, rows1, sidx0, sidx1, didx0, didx1, sem0, sem1):
    c = lax.axis_index("c")
    s = lax.axis_index("s")
    r0 = s * RPT

    pltpu.sync_copy(zeros_hbm, rows0)
    for off, sz in _CHUNKS:
        pltpu.sync_copy(rows0.at[pl.ds(0, sz)],
                        acc_sh.at[pl.ds(r0 + off, sz)])
    plsc.subcore_barrier()

    def run_pipeline(e0, nchunk):
        pltpu.sync_copy(src_hbm.at[pl.ds(e0, CH)], sidx0)
        pltpu.sync_copy(dst_hbm.at[pl.ds(e0, CH)], didx0)
        pltpu.async_copy(table_hbm.at[sidx0], rows0, sem0)

        def body(t, carry):
            b1 = e0 + (2 * t + 1) * CH
            pltpu.sync_copy(src_hbm.at[pl.ds(b1, CH)], sidx1)
            pltpu.sync_copy(dst_hbm.at[pl.ds(b1, CH)], didx1)
            pltpu.async_copy(table_hbm.at[sidx1], rows1, sem1)
            pltpu.make_async_copy(table_hbm.at[sidx0], rows0, sem0).wait()
            pltpu.sync_copy(rows0, acc_sh.at[didx0], add=True)
            b2 = e0 + (2 * t + 2) * CH
            pltpu.sync_copy(src_hbm.at[pl.ds(b2, CH)], sidx0)
            pltpu.sync_copy(dst_hbm.at[pl.ds(b2, CH)], didx0)
            pltpu.async_copy(table_hbm.at[sidx0], rows0, sem0)
            pltpu.make_async_copy(table_hbm.at[sidx1], rows1, sem1).wait()
            pltpu.sync_copy(rows1, acc_sh.at[didx1], add=True)
            return carry

        lax.fori_loop(0, nchunk // 2 - 1, body, 0)
        b1 = e0 + (nchunk - 1) * CH
        pltpu.sync_copy(src_hbm.at[pl.ds(b1, CH)], sidx1)
        pltpu.sync_copy(dst_hbm.at[pl.ds(b1, CH)], didx1)
        pltpu.async_copy(table_hbm.at[sidx1], rows1, sem1)
        pltpu.make_async_copy(table_hbm.at[sidx0], rows0, sem0).wait()
        pltpu.sync_copy(rows0, acc_sh.at[didx0], add=True)
        pltpu.make_async_copy(table_hbm.at[sidx1], rows1, sem1).wait()
        pltpu.sync_copy(rows1, acc_sh.at[didx1], add=True)

    @pl.when(c == 0)
    def _():
        run_pipeline(s * (NCH0 * CH), NCH0)

    @pl.when(c == 1)
    def _():
        run_pipeline(NS * (NCH0 * CH) + s * (NCH1 * CH), NCH1)

    plsc.subcore_barrier()

    for off, sz in _CHUNKS:
        pltpu.sync_copy(acc_sh.at[pl.ds(r0 + off, sz)],
                        rows0.at[pl.ds(0, sz)])
        pltpu.sync_copy(rows0.at[pl.ds(0, sz)],
                        agg_out.at[c, pl.ds(r0 + off, sz)])


@functools.partial(
    pl.kernel, mesh=_MESH,
    out_type=jax.ShapeDtypeStruct((NC, NP, D), jnp.float32),
    scratch_types=[
        pltpu.VMEM_SHARED((NP, D), jnp.float32),
        pltpu.VMEM((CH, D), jnp.float32),
        pltpu.VMEM((CH,), jnp.int32),
        pltpu.VMEM((CH,), jnp.int32),
        pltpu.SemaphoreType.DMA,
        pltpu.SemaphoreType.DMA,
    ])
def _sc_cnt(dst_hbm, zeros_hbm, ones_hbm, cnt_out,
            acc_sh, ones_v, didx0, didx1, sem0, sem1):
    c = lax.axis_index("c")
    s = lax.axis_index("s")
    wid = s * NC + c
    r0 = s * RPT
    e0 = wid * EPW

    pltpu.sync_copy(zeros_hbm, ones_v)
    for off, sz in _CHUNKS:
        pltpu.sync_copy(ones_v.at[pl.ds(0, sz)],
                        acc_sh.at[pl.ds(r0 + off, sz)])
    pltpu.sync_copy(ones_hbm, ones_v)
    plsc.subcore_barrier()

    pltpu.sync_copy(dst_hbm.at[pl.ds(e0, CH)], didx0)

    def body(t, carry):
        b1 = e0 + (2 * t + 1) * CH
        pltpu.async_copy(dst_hbm.at[pl.ds(b1, CH)], didx1, sem1)
        pltpu.sync_copy(ones_v, acc_sh.at[didx0], add=True)
        pltpu.make_async_copy(dst_hbm.at[pl.ds(b1, CH)], didx1, sem1).wait()
        b2 = e0 + (2 * t + 2) * CH
        pltpu.async_copy(dst_hbm.at[pl.ds(b2, CH)], didx0, sem0)
        pltpu.sync_copy(ones_v, acc_sh.at[didx1], add=True)
        pltpu.make_async_copy(dst_hbm.at[pl.ds(b2, CH)], didx0, sem0).wait()
        return carry

    lax.fori_loop(0, NCHUNK // 2 - 1, body, 0)
    b1 = e0 + (NCHUNK - 1) * CH
    pltpu.async_copy(dst_hbm.at[pl.ds(b1, CH)], didx1, sem1)
    pltpu.sync_copy(ones_v, acc_sh.at[didx0], add=True)
    pltpu.make_async_copy(dst_hbm.at[pl.ds(b1, CH)], didx1, sem1).wait()
    pltpu.sync_copy(ones_v, acc_sh.at[didx1], add=True)
    plsc.subcore_barrier()

    for off, sz in _CHUNKS:
        pltpu.sync_copy(acc_sh.at[pl.ds(r0 + off, sz)],
                        ones_v.at[pl.ds(0, sz)])
        pltpu.sync_copy(ones_v.at[pl.ds(0, sz)],
                        cnt_out.at[c, pl.ds(r0 + off, sz)])


_R = 2000
_GRID = N // _R


def _mm_t(a, w):
    return lax.dot_general(a, w, (((1,), (1,)), ((), ())),
                           preferred_element_type=jnp.float32)


def _sage_body(x_ref, a0_ref, a1_ref, c0_ref, c1_ref, wl_ref, bl_ref,
               wr_ref, o_ref, *, relu):
    cnt = c0_ref[:, 0:1] + c1_ref[:, 0:1]
    rc = 1.0 / jnp.maximum(cnt, 1.0)
    mean = (a0_ref[...] + a1_ref[...]) * rc
    h = _mm_t(mean, wl_ref[...]) + bl_ref[...] + _mm_t(x_ref[...], wr_ref[...])
    o_ref[...] = jnp.maximum(h, 0.0) if relu else h


def _tc_sage(x, a0, a1, c0, c1, wl, bl, wr, relu):
    row = pl.BlockSpec((_R, D), lambda i: (i, 0))
    full = pl.BlockSpec((D, D), lambda i: (0, 0))
    vec = pl.BlockSpec((1, D), lambda i: (0, 0))
    return pl.pallas_call(
        functools.partial(_sage_body, relu=relu),
        grid=(_GRID,),
        in_specs=[row, row, row, row, row, full, vec, full],
        out_specs=row,
        out_shape=jax.ShapeDtypeStruct((N, D), jnp.float32),
    )(x, a0, a1, c0, c1, wl, bl, wr)


def _head1_body(h1_ref, a0_ref, a1_ref, c0_ref, c1_ref, wl_ref, bl_ref,
                wr_ref, wf1_ref, bf1_ref, y_ref, ssum_ref, ssq_ref):
    i = pl.program_id(0)
    cnt = c0_ref[:, 0:1] + c1_ref[:, 0:1]
    rc = 1.0 / jnp.maximum(cnt, 1.0)
    mean = (a0_ref[...] + a1_ref[...]) * rc
    h2 = (_mm_t(mean, wl_ref[...]) + bl_ref[...]
          + _mm_t(h1_ref[...], wr_ref[...]))
    y = _mm_t(h2, wf1_ref[...]) + bf1_ref[...]
    y_ref[...] = y

    @pl.when(i == 0)
    def _():
        ssum_ref[...] = jnp.zeros_like(ssum_ref)
        ssq_ref[...] = jnp.zeros_like(ssq_ref)

    ssum_ref[...] += jnp.sum(y, axis=0, keepdims=True)
    ssq_ref[...] += jnp.sum(y * y, axis=0, keepdims=True)


def _tc_head1(h1, a0, a1, c0, c1, wl, bl, wr, wf1, bf1):
    row = pl.BlockSpec((_R, D), lambda i: (i, 0))
    full = pl.BlockSpec((D, D), lambda i: (0, 0))
    vec = pl.BlockSpec((1, D), lambda i: (0, 0))
    return pl.pallas_call(
        _head1_body,
        grid=(_GRID,),
        in_specs=[row, row, row, row, row, full, vec, full, full, vec],
        out_specs=[row, vec, vec],
        out_shape=[jax.ShapeDtypeStruct((N, D), jnp.float32),
                   jax.ShapeDtypeStruct((1, D), jnp.float32),
                   jax.ShapeDtypeStruct((1, D), jnp.float32)],
        compiler_params=pltpu.CompilerParams(
            dimension_semantics=("arbitrary",)),
    )(h1, a0, a1, c0, c1, wl, bl, wr, wf1, bf1)


def _head2_body(y_ref, ssum_ref, ssq_ref, g_ref, b_ref, wf2_ref, bf2_ref,
                z_ref):
    mu = ssum_ref[...] * (1.0 / N)
    var = ssq_ref[...] * (1.0 / N) - mu * mu
    scale = g_ref[...] * lax.rsqrt(var + 1e-5)
    t = (y_ref[...] - mu) * scale + b_ref[...]
    t = jnp.where(t >= 0.0, t, 0.1 * t)
    z_ref[...] = _mm_t(t, wf2_ref[...]) + bf2_ref[...]


def _tc_head2(y, ssum, ssq, gamma, beta, wf2, bf2):
    row = pl.BlockSpec((_R, D), lambda i: (i, 0))
    full = pl.BlockSpec((D, D), lambda i: (0, 0))
    vec = pl.BlockSpec((1, D), lambda i: (0, 0))
    return pl.pallas_call(
        _head2_body,
        grid=(_GRID,),
        in_specs=[row, vec, vec, vec, vec, full, vec],
        out_specs=row,
        out_shape=jax.ShapeDtypeStruct((N, D), jnp.float32),
    )(y, ssum, ssq, gamma, beta, wf2, bf2)


def kernel(x_dict, edge_index_dict, W_l1, b_l1, W_r1, W_l2, b_l2, W_r2,
           W_f1, b_f1, gamma, beta, W_f2, b_f2):
    x = x_dict
    src = edge_index_dict[0]
    dst = edge_index_dict[1]

    pad = EP - E
    src_p = jnp.concatenate([src, jnp.zeros((pad,), jnp.int32)])
    dst_p = jnp.concatenate([dst, jnp.full((pad,), N, jnp.int32)])
    zeros = jnp.zeros((CH, D), jnp.float32)
    ones = jnp.ones((CH, D), jnp.float32)

    xp = jnp.zeros((NP, D), jnp.float32).at[:N].set(x)
    agg1 = _sc_agg(xp, src_p, dst_p, zeros)
    cnt = _sc_cnt(dst_p, zeros, ones)
    a0, a1 = agg1[0, :N], agg1[1, :N]
    c0, c1 = cnt[0, :N], cnt[1, :N]

    b_l1_2 = b_l1.reshape(1, D)
    b_l2_2 = b_l2.reshape(1, D)
    b_f1_2 = b_f1.reshape(1, D)
    b_f2_2 = b_f2.reshape(1, D)

    h1 = _tc_sage(x, a0, a1, c0, c1, W_l1, b_l1_2, W_r1, relu=True)

    h1p = jnp.zeros((NP, D), jnp.float32).at[:N].set(h1)
    agg2 = _sc_agg(h1p, src_p, dst_p, zeros)
    y, ssum, ssq = _tc_head1(h1, agg2[0, :N], agg2[1, :N], c0, c1,
                             W_l2, b_l2_2, W_r2, W_f1, b_f1_2)
    z = _tc_head2(y, ssum, ssq, gamma.reshape(1, D), beta.reshape(1, D),
                  W_f2, b_f2_2)
    return z


# speedup vs baseline: 1.0940x; 1.0940x over previous
import functools

import jax
import jax.numpy as jnp
from jax import lax
from jax.experimental import pallas as pl
from jax.experimental.pallas import tpu as pltpu
from jax.experimental.pallas import tpu_sc as plsc

N = 10000
E = 320000
D = 128

NC = 2
NS = 16
NW = NC * NS
CH = 128
NCHUNK = 2 * (-(-E // (NW * CH * 2)))
EPW = NCHUNK * CH
EP = EPW * NW
NCH0 = 114
NCH1 = 2 * NCHUNK - NCH0
NP = -(-(N + 1) // (NS * 8)) * NS * 8
RPT = NP // NS
_CHUNKS = [(o, min(CH, RPT - o)) for o in range(0, RPT, CH)]

_MESH = plsc.VectorSubcoreMesh(core_axis_name="c", subcore_axis_name="s")


@functools.partial(
    pl.kernel, mesh=_MESH,
    out_type=jax.ShapeDtypeStruct((NC, NP, D), jnp.float32),
    scratch_types=[
        pltpu.VMEM_SHARED((NP, D), jnp.float32),
        pltpu.VMEM((CH, D), jnp.float32),
        pltpu.VMEM((CH, D), jnp.float32),
        pltpu.VMEM((CH,), jnp.int32),
        pltpu.VMEM((CH,), jnp.int32),
        pltpu.VMEM((CH,), jnp.int32),
        pltpu.VMEM((CH,), jnp.int32),
        pltpu.SemaphoreType.DMA,
        pltpu.SemaphoreType.DMA,
    ])
def _sc_agg(table_hbm, src_hbm, dst_hbm, zeros_hbm, agg_out,
            acc_sh, rows0, rows1, sidx0, sidx1, didx0, didx1, sem0, sem1):
    c = lax.axis_index("c")
    s = lax.axis_index("s")
    r0 = s * RPT

    pltpu.sync_copy(zeros_hbm, rows0)
    for off, sz in _CHUNKS:
        pltpu.sync_copy(rows0.at[pl.ds(0, sz)],
                        acc_sh.at[pl.ds(r0 + off, sz)])
    plsc.subcore_barrier()

    def run_pipeline(e0, nchunk):
        pltpu.sync_copy(src_hbm.at[pl.ds(e0, CH)], sidx0)
        pltpu.sync_copy(dst_hbm.at[pl.ds(e0, CH)], didx0)
        pltpu.async_copy(table_hbm.at[sidx0], rows0, sem0)

        def body(t, carry):
            b1 = e0 + (2 * t + 1) * CH
            pltpu.sync_copy(src_hbm.at[pl.ds(b1, CH)], sidx1)
            pltpu.sync_copy(dst_hbm.at[pl.ds(b1, CH)], didx1)
            pltpu.async_copy(table_hbm.at[sidx1], rows1, sem1)
            pltpu.make_async_copy(table_hbm.at[sidx0], rows0, sem0).wait()
            pltpu.sync_copy(rows0, acc_sh.at[didx0], add=True)
            b2 = e0 + (2 * t + 2) * CH
            pltpu.sync_copy(src_hbm.at[pl.ds(b2, CH)], sidx0)
            pltpu.sync_copy(dst_hbm.at[pl.ds(b2, CH)], didx0)
            pltpu.async_copy(table_hbm.at[sidx0], rows0, sem0)
            pltpu.make_async_copy(table_hbm.at[sidx1], rows1, sem1).wait()
            pltpu.sync_copy(rows1, acc_sh.at[didx1], add=True)
            return carry

        lax.fori_loop(0, nchunk // 2 - 1, body, 0)
        b1 = e0 + (nchunk - 1) * CH
        pltpu.sync_copy(src_hbm.at[pl.ds(b1, CH)], sidx1)
        pltpu.sync_copy(dst_hbm.at[pl.ds(b1, CH)], didx1)
        pltpu.async_copy(table_hbm.at[sidx1], rows1, sem1)
        pltpu.make_async_copy(table_hbm.at[sidx0], rows0, sem0).wait()
        pltpu.sync_copy(rows0, acc_sh.at[didx0], add=True)
        pltpu.make_async_copy(table_hbm.at[sidx1], rows1, sem1).wait()
        pltpu.sync_copy(rows1, acc_sh.at[didx1], add=True)

    @pl.when(c == 0)
    def _():
        run_pipeline(s * (NCH0 * CH), NCH0)

    @pl.when(c == 1)
    def _():
        run_pipeline(NS * (NCH0 * CH) + s * (NCH1 * CH), NCH1)

    plsc.subcore_barrier()

    for off, sz in _CHUNKS:
        pltpu.sync_copy(acc_sh.at[pl.ds(r0 + off, sz)],
                        rows0.at[pl.ds(0, sz)])
        pltpu.sync_copy(rows0.at[pl.ds(0, sz)],
                        agg_out.at[c, pl.ds(r0 + off, sz)])


@functools.partial(
    pl.kernel, mesh=_MESH,
    out_type=jax.ShapeDtypeStruct((NC, NP, D), jnp.float32),
    scratch_types=[
        pltpu.VMEM_SHARED((NP, D), jnp.float32),
        pltpu.VMEM((CH, D), jnp.float32),
        pltpu.VMEM((CH,), jnp.int32),
        pltpu.VMEM((CH,), jnp.int32),
        pltpu.SemaphoreType.DMA,
        pltpu.SemaphoreType.DMA,
    ])
def _sc_cnt(dst_hbm, zeros_hbm, ones_hbm, cnt_out,
            acc_sh, ones_v, didx0, didx1, sem0, sem1):
    c = lax.axis_index("c")
    s = lax.axis_index("s")
    wid = s * NC + c
    r0 = s * RPT
    e0 = wid * EPW

    pltpu.sync_copy(zeros_hbm, ones_v)
    for off, sz in _CHUNKS:
        pltpu.sync_copy(ones_v.at[pl.ds(0, sz)],
                        acc_sh.at[pl.ds(r0 + off, sz)])
    pltpu.sync_copy(ones_hbm, ones_v)
    plsc.subcore_barrier()

    pltpu.sync_copy(dst_hbm.at[pl.ds(e0, CH)], didx0)

    def body(t, carry):
        b1 = e0 + (2 * t + 1) * CH
        pltpu.async_copy(dst_hbm.at[pl.ds(b1, CH)], didx1, sem1)
        pltpu.sync_copy(ones_v, acc_sh.at[didx0], add=True)
        pltpu.make_async_copy(dst_hbm.at[pl.ds(b1, CH)], didx1, sem1).wait()
        b2 = e0 + (2 * t + 2) * CH
        pltpu.async_copy(dst_hbm.at[pl.ds(b2, CH)], didx0, sem0)
        pltpu.sync_copy(ones_v, acc_sh.at[didx1], add=True)
        pltpu.make_async_copy(dst_hbm.at[pl.ds(b2, CH)], didx0, sem0).wait()
        return carry

    lax.fori_loop(0, NCHUNK // 2 - 1, body, 0)
    b1 = e0 + (NCHUNK - 1) * CH
    pltpu.async_copy(dst_hbm.at[pl.ds(b1, CH)], didx1, sem1)
    pltpu.sync_copy(ones_v, acc_sh.at[didx0], add=True)
    pltpu.make_async_copy(dst_hbm.at[pl.ds(b1, CH)], didx1, sem1).wait()
    pltpu.sync_copy(ones_v, acc_sh.at[didx1], add=True)
    plsc.subcore_barrier()

    for off, sz in _CHUNKS:
        pltpu.sync_copy(acc_sh.at[pl.ds(r0 + off, sz)],
                        ones_v.at[pl.ds(0, sz)])
        pltpu.sync_copy(ones_v.at[pl.ds(0, sz)],
                        cnt_out.at[c, pl.ds(r0 + off, sz)])


_R = 2000
_GRID = N // _R


def _mm_t(a, w):
    return lax.dot_general(a, w, (((1,), (1,)), ((), ())),
                           preferred_element_type=jnp.float32)


def _sage_body(x_ref, a0_ref, a1_ref, c0_ref, c1_ref, wl_ref, bl_ref,
               wr_ref, o_ref, *, relu):
    cnt = c0_ref[:, 0:1] + c1_ref[:, 0:1]
    rc = 1.0 / jnp.maximum(cnt, 1.0)
    mean = (a0_ref[...] + a1_ref[...]) * rc
    h = _mm_t(mean, wl_ref[...]) + bl_ref[...] + _mm_t(x_ref[...], wr_ref[...])
    o_ref[...] = jnp.maximum(h, 0.0) if relu else h


def _tc_sage(x, a0, a1, c0, c1, wl, bl, wr, relu):
    row = pl.BlockSpec((_R, D), lambda i: (i, 0))
    full = pl.BlockSpec((D, D), lambda i: (0, 0))
    vec = pl.BlockSpec((1, D), lambda i: (0, 0))
    return pl.pallas_call(
        functools.partial(_sage_body, relu=relu),
        grid=(_GRID,),
        in_specs=[row, row, row, row, row, full, vec, full],
        out_specs=row,
        out_shape=jax.ShapeDtypeStruct((N, D), jnp.float32),
    )(x, a0, a1, c0, c1, wl, bl, wr)


def _head1_body(h1_ref, a0_ref, a1_ref, c0_ref, c1_ref, wl_ref, bl_ref,
                wr_ref, wf1_ref, bf1_ref, y_ref, ssum_ref, ssq_ref):
    i = pl.program_id(0)
    cnt = c0_ref[:, 0:1] + c1_ref[:, 0:1]
    rc = 1.0 / jnp.maximum(cnt, 1.0)
    mean = (a0_ref[...] + a1_ref[...]) * rc
    h2 = (_mm_t(mean, wl_ref[...]) + bl_ref[...]
          + _mm_t(h1_ref[...], wr_ref[...]))
    y = _mm_t(h2, wf1_ref[...]) + bf1_ref[...]
    y_ref[...] = y

    @pl.when(i == 0)
    def _():
        ssum_ref[...] = jnp.zeros_like(ssum_ref)
        ssq_ref[...] = jnp.zeros_like(ssq_ref)

    ssum_ref[...] += jnp.sum(y, axis=0, keepdims=True)
    ssq_ref[...] += jnp.sum(y * y, axis=0, keepdims=True)


def _tc_head1(h1, a0, a1, c0, c1, wl, bl, wr, wf1, bf1):
    row = pl.BlockSpec((_R, D), lambda i: (i, 0))
    full = pl.BlockSpec((D, D), lambda i: (0, 0))
    vec = pl.BlockSpec((1, D), lambda i: (0, 0))
    return pl.pallas_call(
        _head1_body,
        grid=(_GRID,),
        in_specs=[row, row, row, row, row, full, vec, full, full, vec],
        out_specs=[row, vec, vec],
        out_shape=[jax.ShapeDtypeStruct((N, D), jnp.float32),
                   jax.ShapeDtypeStruct((1, D), jnp.float32),
                   jax.ShapeDtypeStruct((1, D), jnp.float32)],
        compiler_params=pltpu.CompilerParams(
            dimension_semantics=("arbitrary",)),
    )(h1, a0, a1, c0, c1, wl, bl, wr, wf1, bf1)


def _head2_body(y_ref, ssum_ref, ssq_ref, g_ref, b_ref, wf2_ref, bf2_ref,
                z_ref):
    mu = ssum_ref[...] * (1.0 / N)
    var = ssq_ref[...] * (1.0 / N) - mu * mu
    scale = g_ref[...] * lax.rsqrt(var + 1e-5)
    t = (y_ref[...] - mu) * scale + b_ref[...]
    t = jnp.where(t >= 0.0, t, 0.1 * t)
    z_ref[...] = _mm_t(t, wf2_ref[...]) + bf2_ref[...]


def _tc_head2(y, ssum, ssq, gamma, beta, wf2, bf2):
    row = pl.BlockSpec((_R, D), lambda i: (i, 0))
    full = pl.BlockSpec((D, D), lambda i: (0, 0))
    vec = pl.BlockSpec((1, D), lambda i: (0, 0))
    return pl.pallas_call(
        _head2_body,
        grid=(_GRID,),
        in_specs=[row, vec, vec, vec, vec, full, vec],
        out_specs=row,
        out_shape=jax.ShapeDtypeStruct((N, D), jnp.float32),
    )(y, ssum, ssq, gamma, beta, wf2, bf2)


def kernel(x_dict, edge_index_dict, W_l1, b_l1, W_r1, W_l2, b_l2, W_r2,
           W_f1, b_f1, gamma, beta, W_f2, b_f2):
    x = x_dict
    src = edge_index_dict[0]
    dst = edge_index_dict[1]

    pad = EP - E
    src_p = jnp.concatenate([src, jnp.zeros((pad,), jnp.int32)])
    dst_p = jnp.concatenate([dst, jnp.full((pad,), N, jnp.int32)])
    zeros = jnp.zeros((CH, D), jnp.float32)
    ones = jnp.ones((CH, D), jnp.float32)

    xp = jnp.zeros((NP, D), jnp.float32).at[:N].set(x)
    agg1 = _sc_agg(xp, src_p, dst_p, zeros)
    cnt = _sc_cnt(dst_p, zeros, ones)
    a0, a1 = agg1[0, :N], agg1[1, :N]
    c0, c1 = cnt[0, :N], cnt[1, :N]

    b_l1_2 = b_l1.reshape(1, D)
    b_l2_2 = b_l2.reshape(1, D)
    b_f1_2 = b_f1.reshape(1, D)
    b_f2_2 = b_f2.reshape(1, D)

    h1 = _tc_sage(x, a0, a1, c0, c1, W_l1, b_l1_2, W_r1, relu=True)

    h1p = jnp.zeros((NP, D), jnp.float32).at[:N].set(h1)
    agg2 = _sc_agg(h1p, src_p, dst_p, zeros)
    y, ssum, ssq = _tc_head1(h1, agg2[0, :N], agg2[1, :N], c0, c1,
                             W_l2, b_l2_2, W_r2, W_f1, b_f1_2)
    z = _tc_head2(y, ssum, ssq, gamma.reshape(1, D), beta.reshape(1, D),
                  W_f2, b_f2_2)
    return z
